# baseline (speedup 1.0000x reference)
import functools

import jax
import jax.numpy as jnp
from jax import lax
from jax.experimental import pallas as pl
from jax.experimental.pallas import tpu as pltpu
from jax.experimental.pallas import tpu_sc as plsc

N = 10000
B = 128
D = 768
KL = 128
NB = 5000
BIG = 2**30


def _match_body(x_ref, inp_ref, out_ref):
    i = pl.program_id(0)
    t = inp_ref[...]
    qt = x_ref[...].T
    tlo = (t & 0xFF).astype(jnp.float32)
    thi = (t >> 8).astype(jnp.float32)
    qtlo = (qt & 0xFF).astype(jnp.float32)
    qthi = (qt >> 8).astype(jnp.float32)
    a = jnp.concatenate([tlo, thi], axis=1)
    bq = jnp.concatenate([qtlo, qthi], axis=0)
    s = lax.dot_general(
        a.astype(jnp.bfloat16), bq.astype(jnp.bfloat16),
        (((1,), (0,)), ((), ())),
        preferred_element_type=jnp.float32)
    na = jnp.sum(a * a, axis=1, keepdims=True)
    nq = jnp.sum(bq * bq, axis=0, keepdims=True)
    dist = (na - s) + (nq - s)
    rown = lax.broadcasted_iota(jnp.int32, (NB, B), 0) + i * NB
    cand = jnp.where(dist == 0.0, rown, BIG)
    colmin = jnp.min(cand, axis=0)

    @pl.when(i == 0)
    def _():
        out_ref[...] = jnp.full((8, B), BIG, jnp.int32)

    out_ref[...] = jnp.minimum(out_ref[...], colmin[None, :])


def _match_idx(x2, inp2, interpret=False):
    return pl.pallas_call(
        _match_body,
        grid=(N // NB,),
        in_specs=[
            pl.BlockSpec((B, KL), lambda i: (0, 0)),
            pl.BlockSpec((NB, KL), lambda i: (i, 0)),
        ],
        out_specs=pl.BlockSpec((8, B), lambda i: (0, 0)),
        out_shape=jax.ShapeDtypeStruct((8, B), jnp.int32),
        interpret=interpret,
    )(x2, inp2)


NSTRIP = 10
SR = N // NSTRIP


def _match_body_striped(x_ref, inp_any, out_ref, buf, sems):
    copies = [
        pltpu.make_async_copy(
            inp_any.at[pl.ds(j * SR, SR), :],
            buf.at[pl.ds(j * SR, SR), :],
            sems.at[j],
        )
        for j in range(NSTRIP)
    ]
    for cp in copies:
        cp.start()

    qt = x_ref[...].T
    qtlo = (qt & 0xFF).astype(jnp.float32)
    qthi = (qt >> 8).astype(jnp.float32)
    bq = jnp.concatenate([qtlo, qthi], axis=0)
    nq = jnp.sum(bq * bq, axis=0, keepdims=True)
    bqh = bq.astype(jnp.bfloat16)

    mins = jnp.full((8, B), BIG, jnp.int32)
    for j in range(NSTRIP):
        copies[j].wait()
        t = buf[pl.ds(j * SR, SR), :]
        tlo = (t & 0xFF).astype(jnp.float32)
        thi = (t >> 8).astype(jnp.float32)
        a = jnp.concatenate([tlo, thi], axis=1)
        s = lax.dot_general(
            a.astype(jnp.bfloat16), bqh,
            (((1,), (0,)), ((), ())),
            preferred_element_type=jnp.float32)
        na = jnp.sum(a * a, axis=1, keepdims=True)
        dist = (na - s) + (nq - s)
        rown = lax.broadcasted_iota(jnp.int32, (SR, B), 0) + j * SR
        cand = jnp.where(dist == 0.0, rown, BIG)
        colmin = jnp.min(cand, axis=0)
        mins = jnp.minimum(mins, colmin[None, :])
    out_ref[...] = mins


def _match_idx_striped(x2, inp2, interpret=False):
    return pl.pallas_call(
        _match_body_striped,
        in_specs=[
            pl.BlockSpec(memory_space=pltpu.VMEM),
            pl.BlockSpec(memory_space=pl.ANY),
        ],
        out_specs=pl.BlockSpec(memory_space=pltpu.VMEM),
        out_shape=jax.ShapeDtypeStruct((8, B), jnp.int32),
        scratch_shapes=[
            pltpu.VMEM((N, KL), jnp.int32),
            pltpu.SemaphoreType.DMA((NSTRIP,)),
        ],
        interpret=interpret,
    )(x2, inp2)


_NC = 2
ROWS_PER_W = 8
ACTIVE_W = B // ROWS_PER_W


@functools.cache
def _sc_gather_kernel():
    @functools.partial(
        pl.kernel,
        mesh=plsc.VectorSubcoreMesh(core_axis_name="c", subcore_axis_name="s",
                                    num_cores=1),
        out_type=jax.ShapeDtypeStruct((B, D), jnp.float32),
        scratch_types=[
            pltpu.VMEM((ROWS_PER_W,), jnp.int32),
            pltpu.VMEM((ROWS_PER_W, D), jnp.float32),
            pltpu.SemaphoreType.DMA,
        ],
    )
    def _sc_gather(table_hbm, idx_hbm, out_hbm, idx_v, rows_v, sem):
        wid = lax.axis_index("s")
        base = wid * ROWS_PER_W
        pltpu.sync_copy(idx_hbm.at[0, pl.ds(base, ROWS_PER_W)], idx_v)
        pltpu.async_copy(table_hbm.at[idx_v], rows_v, sem).wait()
        pltpu.sync_copy(rows_v, out_hbm.at[pl.ds(base, ROWS_PER_W)])

    return _sc_gather


def kernel(x, inputs, outputs):
    x2 = x.reshape(B, KL)
    inp2 = inputs.reshape(N, KL)
    idx8 = _match_idx_striped(x2, inp2)
    return _sc_gather_kernel()(outputs, idx8)


# speedup vs baseline: 1.0387x; 1.0387x over previous
import functools

import jax
import jax.numpy as jnp
from jax import lax
from jax.experimental import pallas as pl
from jax.experimental.pallas import tpu as pltpu
from jax.experimental.pallas import tpu_sc as plsc

N = 10000
B = 128
D = 768
KL = 128
NB = 5000
BIG = 2**30


def _match_body(x_ref, inp_ref, out_ref):
    i = pl.program_id(0)
    t = inp_ref[...]
    qt = x_ref[...].T
    tlo = (t & 0xFF).astype(jnp.float32)
    thi = (t >> 8).astype(jnp.float32)
    qtlo = (qt & 0xFF).astype(jnp.float32)
    qthi = (qt >> 8).astype(jnp.float32)
    a = jnp.concatenate([tlo, thi], axis=1)
    bq = jnp.concatenate([qtlo, qthi], axis=0)
    s = lax.dot_general(
        a.astype(jnp.bfloat16), bq.astype(jnp.bfloat16),
        (((1,), (0,)), ((), ())),
        preferred_element_type=jnp.float32)
    na = jnp.sum(a * a, axis=1, keepdims=True)
    nq = jnp.sum(bq * bq, axis=0, keepdims=True)
    dist = (na - s) + (nq - s)
    rown = lax.broadcasted_iota(jnp.int32, (NB, B), 0) + i * NB
    cand = jnp.where(dist == 0.0, rown, BIG)
    colmin = jnp.min(cand, axis=0)

    @pl.when(i == 0)
    def _():
        out_ref[...] = jnp.full((8, B), BIG, jnp.int32)

    out_ref[...] = jnp.minimum(out_ref[...], colmin[None, :])


def _match_idx(x2, inp2, interpret=False):
    return pl.pallas_call(
        _match_body,
        grid=(N // NB,),
        in_specs=[
            pl.BlockSpec((B, KL), lambda i: (0, 0)),
            pl.BlockSpec((NB, KL), lambda i: (i, 0)),
        ],
        out_specs=pl.BlockSpec((8, B), lambda i: (0, 0)),
        out_shape=jax.ShapeDtypeStruct((8, B), jnp.int32),
        interpret=interpret,
    )(x2, inp2)


_NC = 2
ROWS_PER_W = 16
ACTIVE_W = B // ROWS_PER_W


@functools.cache
def _sc_gather_kernel():
    @functools.partial(
        pl.kernel,
        mesh=plsc.VectorSubcoreMesh(core_axis_name="c", subcore_axis_name="s",
                                    num_cores=1, num_subcores=8),
        out_type=jax.ShapeDtypeStruct((B, D), jnp.float32),
        scratch_types=[
            pltpu.VMEM((ROWS_PER_W,), jnp.int32),
            pltpu.VMEM((ROWS_PER_W, D), jnp.float32),
            pltpu.SemaphoreType.DMA,
        ],
    )
    def _sc_gather(table_hbm, idx_hbm, out_hbm, idx_v, rows_v, sem):
        wid = lax.axis_index("s")
        base = wid * ROWS_PER_W
        pltpu.sync_copy(idx_hbm.at[0, pl.ds(base, ROWS_PER_W)], idx_v)
        pltpu.async_copy(table_hbm.at[idx_v], rows_v, sem).wait()
        pltpu.sync_copy(rows_v, out_hbm.at[pl.ds(base, ROWS_PER_W)])

    return _sc_gather


def kernel(x, inputs, outputs):
    x2 = x.reshape(B, KL)
    inp2 = inputs.reshape(N, KL)
    idx8 = _match_idx(x2, inp2)
    return _sc_gather_kernel()(outputs, idx8)


# speedup vs baseline: 1.0785x; 1.0383x over previous
import functools

import jax
import jax.numpy as jnp
from jax import lax
from jax.experimental import pallas as pl
from jax.experimental.pallas import tpu as pltpu
from jax.experimental.pallas import tpu_sc as plsc

N = 10000
B = 128
D = 768
KL = 128
NB = 2000
BIG = 2**30


def _match_body(x_ref, inp_ref, out_ref):
    i = pl.program_id(0)
    t = inp_ref[...].astype(jnp.int32)
    qt = x_ref[...].T.astype(jnp.int32)
    tlo = (t & 0xFF).astype(jnp.float32)
    thi = ((t >> 8) & 0xFF).astype(jnp.float32)
    qtlo = (qt & 0xFF).astype(jnp.float32)
    qthi = ((qt >> 8) & 0xFF).astype(jnp.float32)
    a = jnp.concatenate([tlo, thi], axis=1)
    bq = jnp.concatenate([qtlo, qthi], axis=0)
    s = lax.dot_general(
        a.astype(jnp.bfloat16), bq.astype(jnp.bfloat16),
        (((1,), (0,)), ((), ())),
        preferred_element_type=jnp.float32)
    na = jnp.sum(a * a, axis=1, keepdims=True)
    nq = jnp.sum(bq * bq, axis=0, keepdims=True)
    dist = (na - s) + (nq - s)
    rown = lax.broadcasted_iota(jnp.int32, (NB, B), 0) + i * NB
    cand = jnp.where(dist == 0.0, rown, BIG)
    colmin = jnp.min(cand, axis=0)

    @pl.when(i == 0)
    def _():
        out_ref[...] = jnp.full((8, B), BIG, jnp.int32)

    out_ref[...] = jnp.minimum(out_ref[...], colmin[None, :])


def _match_idx(x2, inp2, interpret=False):
    return pl.pallas_call(
        _match_body,
        grid=(N // NB,),
        in_specs=[
            pl.BlockSpec((B, KL), lambda i: (0, 0)),
            pl.BlockSpec((NB, KL), lambda i: (i, 0)),
        ],
        out_specs=pl.BlockSpec((8, B), lambda i: (0, 0)),
        out_shape=jax.ShapeDtypeStruct((8, B), jnp.int32),
        interpret=interpret,
    )(x2, inp2)


_NC = 2
ROWS_PER_W = 8
ACTIVE_W = B // ROWS_PER_W


@functools.cache
def _sc_gather_kernel():
    @functools.partial(
        pl.kernel,
        mesh=plsc.VectorSubcoreMesh(core_axis_name="c", subcore_axis_name="s",
                                    num_cores=1),
        out_type=jax.ShapeDtypeStruct((B, D), jnp.float32),
        scratch_types=[
            pltpu.VMEM((ROWS_PER_W,), jnp.int32),
            pltpu.VMEM((ROWS_PER_W, D), jnp.float32),
            pltpu.SemaphoreType.DMA,
        ],
    )
    def _sc_gather(table_hbm, idx_hbm, out_hbm, idx_v, rows_v, sem):
        wid = lax.axis_index("s")
        base = wid * ROWS_PER_W
        pltpu.sync_copy(idx_hbm.at[0, pl.ds(base, ROWS_PER_W)], idx_v)
        pltpu.async_copy(table_hbm.at[idx_v], rows_v, sem).wait()
        pltpu.sync_copy(rows_v, out_hbm.at[pl.ds(base, ROWS_PER_W)])

    return _sc_gather


def kernel(x, inputs, outputs):
    x2 = x.reshape(B, KL).astype(jnp.int16)
    inp2 = inputs.reshape(N, KL).astype(jnp.int16)
    idx8 = _match_idx(x2, inp2)
    return _sc_gather_kernel()(outputs, idx8)


# speedup vs baseline: 1.0979x; 1.0180x over previous
import functools

import jax
import jax.numpy as jnp
from jax import lax
from jax.experimental import pallas as pl
from jax.experimental.pallas import tpu as pltpu
from jax.experimental.pallas import tpu_sc as plsc

N = 10000
B = 128
D = 768
KL = 128
NB = 10000
BIG = 2**30


def _match_body(x_ref, inp_ref, out_ref):
    i = pl.program_id(0)
    t = inp_ref[...].astype(jnp.int32)
    qt = x_ref[...].T.astype(jnp.int32)
    tlo = (t & 0xFF).astype(jnp.float32)
    thi = ((t >> 8) & 0xFF).astype(jnp.float32)
    qtlo = (qt & 0xFF).astype(jnp.float32)
    qthi = ((qt >> 8) & 0xFF).astype(jnp.float32)
    a = jnp.concatenate([tlo, thi], axis=1)
    bq = jnp.concatenate([qtlo, qthi], axis=0)
    s = lax.dot_general(
        a.astype(jnp.bfloat16), bq.astype(jnp.bfloat16),
        (((1,), (0,)), ((), ())),
        preferred_element_type=jnp.float32)
    na = jnp.sum(a * a, axis=1, keepdims=True)
    nq = jnp.sum(bq * bq, axis=0, keepdims=True)
    dist = (na - s) + (nq - s)
    rown = lax.broadcasted_iota(jnp.int32, (NB, B), 0) + i * NB
    cand = jnp.where(dist == 0.0, rown, BIG)
    colmin = jnp.min(cand, axis=0)

    @pl.when(i == 0)
    def _():
        out_ref[...] = jnp.full((8, B), BIG, jnp.int32)

    out_ref[...] = jnp.minimum(out_ref[...], colmin[None, :])


def _match_idx(x2, inp2, interpret=False):
    return pl.pallas_call(
        _match_body,
        grid=(N // NB,),
        in_specs=[
            pl.BlockSpec((B, KL), lambda i: (0, 0)),
            pl.BlockSpec((NB, KL), lambda i: (i, 0)),
        ],
        out_specs=pl.BlockSpec((8, B), lambda i: (0, 0)),
        out_shape=jax.ShapeDtypeStruct((8, B), jnp.int32),
        interpret=interpret,
    )(x2, inp2)


_NC = 2
ROWS_PER_W = 8
ACTIVE_W = B // ROWS_PER_W


@functools.cache
def _sc_gather_kernel():
    @functools.partial(
        pl.kernel,
        mesh=plsc.VectorSubcoreMesh(core_axis_name="c", subcore_axis_name="s",
                                    num_cores=1),
        out_type=jax.ShapeDtypeStruct((B, D), jnp.float32),
        scratch_types=[
            pltpu.VMEM((ROWS_PER_W,), jnp.int32),
            pltpu.VMEM((ROWS_PER_W, D), jnp.float32),
            pltpu.SemaphoreType.DMA,
        ],
    )
    def _sc_gather(table_hbm, idx_hbm, out_hbm, idx_v, rows_v, sem):
        wid = lax.axis_index("s")
        base = wid * ROWS_PER_W
        pltpu.sync_copy(idx_hbm.at[0, pl.ds(base, ROWS_PER_W)], idx_v)
        pltpu.async_copy(table_hbm.at[idx_v], rows_v, sem).wait()
        pltpu.sync_copy(rows_v, out_hbm.at[pl.ds(base, ROWS_PER_W)])

    return _sc_gather


def kernel(x, inputs, outputs):
    x2 = x.reshape(B, KL).astype(jnp.int16)
    inp2 = inputs.reshape(N, KL).astype(jnp.int16)
    idx8 = _match_idx(x2, inp2)
    return _sc_gather_kernel()(outputs, idx8)
